# Initial kernel scaffold; baseline (speedup 1.0000x reference)
import jax
import jax.numpy as jnp
from jax.experimental import pallas as pl


def kernel(x, token_table, position_table):
    raise NotImplementedError("write your pallas kernel here")


# speedup vs baseline: 3.9306x; 3.9306x over previous
import functools

import jax
import jax.numpy as jnp
from jax import lax
from jax.experimental import pallas as pl
from jax.experimental.pallas import tpu as pltpu
from jax.experimental.pallas import tpu_sc as plsc

VOCAB = 100000
SEQ = 200
DIM = 128
BATCH = 1024

_INFO = plsc.get_sparse_core_info()
_NC = _INFO.num_cores
_NS = _INFO.num_subcores
_NW = _NC * _NS
_ROWS_PER_W = BATCH // _NW

_CHUNKS = ((0, 104), (104, 96))

_LANES = 16
_VECS_PER_LINE = DIM // _LANES


def _body(x_hbm, tok_hbm, pos_hbm, out_hbm, idx_v, pos_v, buf_v, sem):
    wid = lax.axis_index("s") * _NC + lax.axis_index("c")
    row0 = wid * _ROWS_PER_W

    pltpu.sync_copy(pos_hbm, pos_v)

    @pl.loop(0, _ROWS_PER_W)
    def _row(r):
        base = (row0 + r) * SEQ
        pltpu.sync_copy(x_hbm.at[pl.ds(base, SEQ)], idx_v)
        for off, n in _CHUNKS:
            pltpu.async_copy(
                tok_hbm.at[idx_v.at[pl.ds(off, n)]],
                buf_v.at[pl.ds(off, n), :],
                sem,
            )
        for off, n in _CHUNKS:
            pltpu.make_async_copy(
                tok_hbm.at[idx_v.at[pl.ds(off, n)]],
                buf_v.at[pl.ds(off, n), :],
                sem,
            ).wait()

        @pl.loop(0, SEQ)
        def _line(i):
            for j in range(_VECS_PER_LINE):
                sl = pl.ds(j * _LANES, _LANES)
                buf_v[i, sl] = buf_v[i, sl] + pos_v[i, sl]

        pltpu.sync_copy(buf_v, out_hbm.at[pl.ds(base, SEQ)])


@jax.jit
def _run(x_flat, token_table, position_table):
    mesh = plsc.VectorSubcoreMesh(core_axis_name="c", subcore_axis_name="s")
    return pl.kernel(
        _body,
        out_type=jax.ShapeDtypeStruct((BATCH * SEQ, DIM), jnp.float32),
        mesh=mesh,
        scratch_types=[
            pltpu.VMEM((SEQ,), jnp.int32),
            pltpu.VMEM((SEQ, DIM), jnp.float32),
            pltpu.VMEM((SEQ, DIM), jnp.float32),
            pltpu.SemaphoreType.DMA,
        ],
    )(x_flat, token_table, position_table)


def kernel(x, token_table, position_table):
    x_flat = x.reshape(-1).astype(jnp.int32)
    out = _run(x_flat, token_table, position_table)
    return out.reshape(x.shape[0], x.shape[1], DIM)


# speedup vs baseline: 7.2121x; 1.8349x over previous
import jax
import jax.numpy as jnp
from jax import lax
from jax.experimental import pallas as pl
from jax.experimental.pallas import tpu as pltpu
from jax.experimental.pallas import tpu_sc as plsc

VOCAB = 100000
SEQ = 200
DIM = 128
BATCH = 1024

_INFO = plsc.get_sparse_core_info()
_NC = _INFO.num_cores
_NS = _INFO.num_subcores
_NW = _NC * _NS
_ROWS_PER_W = BATCH // _NW

_CHUNKS = ((0, 104), (104, 96))

_LANES = 16
_VECS_PER_LINE = DIM // _LANES
_NBUF = 3


def _body(x_hbm, tok_hbm, pos_hbm, out_hbm,
          idx_v, pos_v, buf0, buf1, buf2,
          g0, g1, g2, s0, s1, s2):
    bufs = (buf0, buf1, buf2)
    gsems = (g0, g1, g2)
    ssems = (s0, s1, s2)

    wid = lax.axis_index("s") * _NC + lax.axis_index("c")
    row0 = wid * _ROWS_PER_W
    base0 = row0 * SEQ

    pltpu.sync_copy(pos_hbm, pos_v)
    pltpu.sync_copy(x_hbm.at[pl.ds(base0, _ROWS_PER_W * SEQ)], idx_v)

    def gather(r):
        b = r % _NBUF
        for off, n in _CHUNKS:
            pltpu.async_copy(
                tok_hbm.at[idx_v.at[pl.ds(r * SEQ + off, n)]],
                bufs[b].at[pl.ds(off, n), :],
                gsems[b],
            )

    def wait_gather(r):
        b = r % _NBUF
        for off, n in _CHUNKS:
            pltpu.make_async_copy(
                tok_hbm.at[idx_v.at[pl.ds(r * SEQ + off, n)]],
                bufs[b].at[pl.ds(off, n), :],
                gsems[b],
            ).wait()

    def store(r):
        b = r % _NBUF
        pltpu.async_copy(bufs[b], out_hbm.at[pl.ds(base0 + r * SEQ, SEQ)],
                         ssems[b])

    def wait_store(r):
        b = r % _NBUF
        pltpu.make_async_copy(bufs[b],
                              out_hbm.at[pl.ds(base0 + r * SEQ, SEQ)],
                              ssems[b]).wait()

    gather(0)
    for r in range(_ROWS_PER_W):
        if r + 1 < _ROWS_PER_W:
            if r >= 2:
                wait_store(r - 2)
            gather(r + 1)
        wait_gather(r)
        buf = bufs[r % _NBUF]

        @pl.loop(0, SEQ)
        def _line(i):
            for j in range(_VECS_PER_LINE):
                sl = pl.ds(j * _LANES, _LANES)
                buf[i, sl] = buf[i, sl] + pos_v[i, sl]

        store(r)
    for r in range(_ROWS_PER_W - 3, _ROWS_PER_W):
        wait_store(r)


@jax.jit
def _run(x_flat, token_table, position_table):
    mesh = plsc.VectorSubcoreMesh(core_axis_name="c", subcore_axis_name="s")
    return pl.kernel(
        _body,
        out_type=jax.ShapeDtypeStruct((BATCH * SEQ, DIM), jnp.float32),
        mesh=mesh,
        scratch_types=[
            pltpu.VMEM((_ROWS_PER_W * SEQ,), jnp.int32),
            pltpu.VMEM((SEQ, DIM), jnp.float32),
            pltpu.VMEM((SEQ, DIM), jnp.float32),
            pltpu.VMEM((SEQ, DIM), jnp.float32),
            pltpu.VMEM((SEQ, DIM), jnp.float32),
            pltpu.SemaphoreType.DMA,
            pltpu.SemaphoreType.DMA,
            pltpu.SemaphoreType.DMA,
            pltpu.SemaphoreType.DMA,
            pltpu.SemaphoreType.DMA,
            pltpu.SemaphoreType.DMA,
        ],
    )(x_flat, token_table, position_table)


def kernel(x, token_table, position_table):
    x_flat = x.reshape(-1).astype(jnp.int32)
    out = _run(x_flat, token_table, position_table)
    return out.reshape(x.shape[0], x.shape[1], DIM)
